# Initial kernel scaffold; baseline (speedup 1.0000x reference)
import jax
import jax.numpy as jnp
from jax.experimental import pallas as pl


def kernel(coords, input_feat, seps):
    raise NotImplementedError("write your pallas kernel here")


# speedup vs baseline: 2.8648x; 2.8648x over previous
import jax
import jax.numpy as jnp
from jax.experimental import pallas as pl

SUBBUCK = 2
BM = 640


def _pool_body(feat_ref, coord_ref, rfeat_ref, rcoord_ref, unpool_ref):
    f = feat_ref[...]
    d = f.shape[1] // 2
    rfeat_ref[...] = jnp.maximum(f[:, :d], f[:, d:])
    c = coord_ref[...]
    rcoord_ref[...] = (c[:, :3] + c[:, 3:]) * 0.5
    base = pl.program_id(0) * rfeat_ref.shape[0]
    unpool_ref[...] = base + jax.lax.broadcasted_iota(
        jnp.int32, unpool_ref.shape, 0
    )


def kernel(coords, input_feat, seps):
    n, d = input_feat.shape
    rn = n // SUBBUCK
    feat2 = input_feat.reshape(rn, SUBBUCK * d)
    coord2 = coords.reshape(rn, SUBBUCK * 3)
    grid = (rn // BM,)
    rfeat, rcoord, unpool2 = pl.pallas_call(
        _pool_body,
        grid=grid,
        in_specs=[
            pl.BlockSpec((BM, SUBBUCK * d), lambda i: (i, 0)),
            pl.BlockSpec((BM, SUBBUCK * 3), lambda i: (i, 0)),
        ],
        out_specs=[
            pl.BlockSpec((BM, d), lambda i: (i, 0)),
            pl.BlockSpec((BM, 3), lambda i: (i, 0)),
            pl.BlockSpec((BM, SUBBUCK), lambda i: (i, 0)),
        ],
        out_shape=[
            jax.ShapeDtypeStruct((rn, d), input_feat.dtype),
            jax.ShapeDtypeStruct((rn, 3), coords.dtype),
            jax.ShapeDtypeStruct((rn, SUBBUCK), jnp.int32),
        ],
    )(feat2, coord2)
    reduced_sep = (seps + SUBBUCK - 1) // SUBBUCK
    return (rfeat, rcoord, reduced_sep, unpool2.reshape(n))
